# Initial kernel scaffold; baseline (speedup 1.0000x reference)
import jax
import jax.numpy as jnp
from jax.experimental import pallas as pl


def kernel(x, Wq, bq, Wk, bk, Wv, bv, Wo, bo):
    raise NotImplementedError("write your pallas kernel here")


# speedup vs baseline: 10.5113x; 10.5113x over previous
import functools
import math

import jax
import jax.numpy as jnp
import numpy as np
from jax.experimental import pallas as pl

_N_HEAD = 16
_CHUNK = 32
_TOPK = 8
_NEG = -1e30


def _bf(a):
    return a.astype(jnp.bfloat16)


def _mm_bias_body(x_ref, w_ref, b_ref, o_ref):
    acc = jax.lax.dot_general(
        _bf(x_ref[...]), _bf(w_ref[...]), (((1,), (0,)), ((), ())),
        preferred_element_type=jnp.float32)
    o_ref[...] = acc + b_ref[...]


def _mm_bias(x, wt, b, bm, bn):
    M, K = x.shape
    N = wt.shape[1]
    grid = (M // bm, N // bn)
    return pl.pallas_call(
        _mm_bias_body,
        grid=grid,
        in_specs=[
            pl.BlockSpec((bm, K), lambda i, j: (i, 0)),
            pl.BlockSpec((K, bn), lambda i, j: (0, j)),
            pl.BlockSpec((1, bn), lambda i, j: (0, j)),
        ],
        out_specs=pl.BlockSpec((bm, bn), lambda i, j: (i, j)),
        out_shape=jax.ShapeDtypeStruct((M, N), jnp.float32),
    )(x, wt, b.reshape(1, N))


def _top8_mask(s):
    qt, n = s.shape
    iota = jax.lax.broadcasted_iota(jnp.int32, (qt, n), 1)
    work = s
    mask = jnp.zeros((qt, n), dtype=jnp.float32)
    for _ in range(_TOPK):
        mx = jnp.max(work, axis=1, keepdims=True)
        is_mx = work == mx
        first = jnp.min(jnp.where(is_mx, iota, n), axis=1, keepdims=True)
        hit = iota == first
        mask = jnp.where(hit, 1.0, mask)
        work = jnp.where(hit, _NEG, work)
    return mask


def _attn_body(q_ref, ksel_ref, vsel_ref, o_ref, *, qt, skv):
    q = q_ref[0]
    ksel = ksel_ref[0]
    vsel = vsel_ref[0]
    dh = q.shape[1]

    s_all = jax.lax.dot_general(
        _bf(q), _bf(ksel), (((1,), (1,)), ((), ())),
        preferred_element_type=jnp.float32)

    n5 = skv // 2
    s5 = s_all[:, 0:n5]
    s5p = s_all[:, n5:2 * n5]
    s4 = s_all[:, 2 * n5:2 * n5 + n5 // 2]
    s3 = s_all[:, 2 * n5 + n5 // 2:2 * n5 + n5 // 2 + n5 // 4]
    base3 = 2 * n5 + n5 // 2 + n5 // 4
    s2 = s_all[:, base3:base3 + n5 // 8]
    s1 = s_all[:, base3 + n5 // 8:base3 + n5 // 8 + n5 // 16]

    nc = skv // _CHUNK

    def grp(m, g):
        return m[:, g * nc:(g + 1) * nc]

    m1 = _top8_mask(s1)
    p2 = jnp.concatenate([m1, m1], axis=1)
    m2 = _top8_mask(jnp.where(p2 > 0.5, s2, _NEG))
    p3 = jnp.concatenate(
        [grp(m2, 0), grp(m2, 0), grp(m2, 1), grp(m2, 1)], axis=1)
    m3 = _top8_mask(jnp.where(p3 > 0.5, s3, _NEG))
    p4 = jnp.concatenate(
        [grp(m3, g // 2) for g in range(8)], axis=1)
    m4 = _top8_mask(jnp.where(p4 > 0.5, s4, _NEG))
    p5 = jnp.concatenate(
        [grp(m4, g // 2) for g in range(16)], axis=1)
    m5 = _top8_mask(jnp.where(p5 > 0.5, s5, _NEG))

    inv = 1.0 / math.sqrt(dh)
    sel = m5 > 0.5
    l5 = jnp.where(sel, s5 * inv, _NEG)
    l5p = jnp.where(sel, s5p * inv, _NEG)
    mx = jnp.maximum(jnp.max(l5, axis=1, keepdims=True),
                     jnp.max(l5p, axis=1, keepdims=True))
    e5 = jnp.exp(l5 - mx)
    e5p = jnp.exp(l5p - mx)
    den = (jnp.sum(e5, axis=1, keepdims=True)
           + jnp.sum(e5p, axis=1, keepdims=True))
    probs = jnp.concatenate([e5, e5p], axis=1) / den

    ctx = jax.lax.dot_general(
        _bf(probs), _bf(vsel), (((1,), (0,)), ((), ())),
        preferred_element_type=jnp.float32)
    o_ref[0] = ctx


def _rep_perms(skv):
    perm1 = np.arange(_CHUNK // 2, skv, _CHUNK)
    perm2 = np.concatenate(
        [np.arange(o, skv, _CHUNK) for o in (8, 24)])
    perm3 = np.concatenate(
        [np.arange(o, skv, _CHUNK) for o in (4, 12, 20, 28)])
    perm4 = np.concatenate(
        [np.arange(o, skv, _CHUNK) for o in (2, 6, 10, 14, 18, 22, 26, 30)])
    perm5 = np.concatenate(
        [np.arange(o, skv, _CHUNK) for o in range(1, 32, 2)])
    return perm1, perm2, perm3, perm4, perm5


def kernel(x, Wq, bq, Wk, bk, Wv, bv, Wo, bo):
    B, S, D = x.shape
    H = _N_HEAD
    dh = D // H
    x2 = x.reshape(B * S, D)

    wcat = jnp.concatenate([Wq, Wk, Wv], axis=0).T
    bcat = jnp.concatenate([bq, bk, bv], axis=0)
    qkv = _mm_bias(x2, wcat, bcat, 256, 512)
    q = qkv[:, :D].reshape(S, H, dh).transpose(1, 0, 2)
    k = qkv[:, D:2 * D].reshape(S, H, dh).transpose(1, 0, 2)
    v = qkv[:, 2 * D:].reshape(S, H, dh).transpose(1, 0, 2)

    p1, p2, p3, p4, p5 = _rep_perms(S)
    ksel = jnp.concatenate(
        [k[:, p5, :], k[:, p5 - 1, :], k[:, p4, :], k[:, p3, :],
         k[:, p2, :], k[:, p1, :],
         jnp.zeros((H, 64, dh), jnp.float32)], axis=1)
    vsel = jnp.concatenate([v[:, p5, :], v[:, p5 - 1, :]], axis=1)

    qt = 256
    nsel = ksel.shape[1]
    body = functools.partial(_attn_body, qt=qt, skv=S)
    ctx = pl.pallas_call(
        body,
        grid=(H, S // qt),
        in_specs=[
            pl.BlockSpec((1, qt, dh), lambda h, t: (h, t, 0)),
            pl.BlockSpec((1, nsel, dh), lambda h, t: (h, 0, 0)),
            pl.BlockSpec((1, S, dh), lambda h, t: (h, 0, 0)),
        ],
        out_specs=pl.BlockSpec((1, qt, dh), lambda h, t: (h, t, 0)),
        out_shape=jax.ShapeDtypeStruct((H, S, dh), jnp.float32),
    )(q, ksel, vsel)

    ctx2 = ctx.transpose(1, 0, 2).reshape(S, D)
    out = _mm_bias(ctx2, Wo.T, bo, 256, 512)
    return out.reshape(B, S, D)


# speedup vs baseline: 11.0486x; 1.0511x over previous
import functools
import math

import jax
import jax.numpy as jnp
import numpy as np
from jax import lax
from jax.experimental import pallas as pl
from jax.experimental.pallas import tpu as pltpu
from jax.experimental.pallas import tpu_sc as plsc

_N_HEAD = 16
_SEQ = 2048
_DH = 64
_CHUNK = 32
_TOPK = 8
_LANES = 16

_ROW = 3072
_OFF5, _OFF5P, _OFF4, _OFF3, _OFF2, _OFF1 = 0, 1024, 2048, 2560, 2816, 2944

_NW = 32
_ITEMS = _N_HEAD * _SEQ
_PER_W = _ITEMS // _NW
_BLK = 8


def _bf(a):
    return a.astype(jnp.bfloat16)


def _mm_bias_body(x_ref, w_ref, b_ref, o_ref):
    acc = lax.dot_general(
        _bf(x_ref[...]), _bf(w_ref[...]), (((1,), (0,)), ((), ())),
        preferred_element_type=jnp.float32)
    o_ref[...] = acc + b_ref[...]


def _mm_bias(x, wt, b, bm, bn):
    M, K = x.shape
    N = wt.shape[1]
    grid = (M // bm, N // bn)
    return pl.pallas_call(
        _mm_bias_body,
        grid=grid,
        in_specs=[
            pl.BlockSpec((bm, K), lambda i, j: (i, 0)),
            pl.BlockSpec((K, bn), lambda i, j: (0, j)),
            pl.BlockSpec((1, bn), lambda i, j: (0, j)),
        ],
        out_specs=pl.BlockSpec((bm, bn), lambda i, j: (i, j)),
        out_shape=jax.ShapeDtypeStruct((M, N), jnp.float32),
    )(x, wt, b.reshape(1, N))


def _score_body(q_ref, k_ref, o_ref):
    o_ref[0] = lax.dot_general(
        _bf(q_ref[0]), _bf(k_ref[0]), (((1,), (1,)), ((), ())),
        preferred_element_type=jnp.float32)


def _scores(q, ksel, qt=256):
    H, S, dh = q.shape
    n = ksel.shape[1]
    return pl.pallas_call(
        _score_body,
        grid=(H, S // qt),
        in_specs=[
            pl.BlockSpec((1, qt, dh), lambda h, t: (h, t, 0)),
            pl.BlockSpec((1, n, dh), lambda h, t: (h, 0, 0)),
        ],
        out_specs=pl.BlockSpec((1, qt, n), lambda h, t: (h, t, 0)),
        out_shape=jax.ShapeDtypeStruct((H, S, n), jnp.float32),
    )(q, ksel)


def _lane():
    return lax.iota(jnp.int32, _LANES)


def _splat(v):
    return jnp.full((_LANES,), v, jnp.int32)


def _perm(x, idx):
    return lax.gather(
        x, idx.reshape(_LANES, 1),
        lax.GatherDimensionNumbers(
            offset_dims=(), collapsed_slice_dims=(0,), start_index_map=(0,)),
        (1,), mode=lax.GatherScatterMode.PROMISE_IN_BOUNDS)


def _merge8(sa, ia, sb, ib):
    lane = _lane()
    low = lane < 8
    l8 = lane & 7
    s = jnp.where(low, sa, _perm(sb, l8))
    i = jnp.where(low, ia, _perm(ib, l8))
    return plsc.sort_key_val(s, i, descending=True)


def _children(p):
    lane = _lane()
    base = (p & 63) | ((p >> 6) << 7)
    return _perm(base, lane & 7) + jnp.where(lane < 8, 0, 64)


def _sc_attend_body(sall, vtab, out, sbuf, rows, idxv, obuf, sem):
    wid = lax.axis_index("s") * 2 + lax.axis_index("c")
    head = wid // 2
    base = wid * _PER_W
    lane = _lane()
    low = lane < 8
    l8 = lane & 7

    def blk_body(b, carry):
        it0 = base + b * _BLK
        pltpu.sync_copy(sall.at[pl.ds(it0 * _ROW, _BLK * _ROW)], sbuf)

        def item_body(i, carry2):
            rbase = i * _ROW

            svs = []
            for kk in range(4):
                s = plsc.load_gather(sbuf, [_splat(rbase + _OFF1 + 16 * kk) + lane])
                svs.append(plsc.sort_key_val(s, _splat(16 * kk) + lane,
                                             descending=True))
            s01 = _merge8(*svs[0], *svs[1])
            s23 = _merge8(*svs[2], *svs[3])
            _, p = _merge8(*s01, *s23)

            for off in (_OFF2, _OFF3, _OFF4, _OFF5):
                cand = _children(p)
                s = plsc.load_gather(sbuf, [_splat(rbase + off) + cand])
                sv, p = plsc.sort_key_val(s, cand, descending=True)

            part = plsc.load_gather(sbuf, [_splat(rbase + _OFF5P) + p])
            fs = jnp.where(low, sv, _perm(part, l8))
            fidx = jnp.where(low, p, _perm(p, l8) + 1024)

            x = fs * (1.0 / math.sqrt(_DH))
            e = jnp.exp(x - jnp.max(x))
            w = e / jnp.sum(e)

            idxv[...] = fidx + head * _SEQ
            pltpu.async_copy(vtab.at[idxv], rows, sem).wait()

            acc = [jnp.zeros((_LANES,), jnp.float32) for _ in range(4)]
            for r in range(16):
                wr = _perm(w, _splat(r))
                for g in range(4):
                    v = rows[r, pl.ds(16 * g, 16)]
                    acc[g] = acc[g] + v * wr
            for g in range(4):
                plsc.store_scatter(obuf, [_splat(i * _DH + 16 * g) + lane],
                                   acc[g])
            return carry2

        lax.fori_loop(0, _BLK, item_body, 0)
        pltpu.sync_copy(obuf, out.at[pl.ds(it0 * _DH, _BLK * _DH)])
        return carry

    lax.fori_loop(0, _PER_W // _BLK, blk_body, 0)


def _sc_attend(sall, vtab):
    mesh = plsc.VectorSubcoreMesh(core_axis_name="c", subcore_axis_name="s")
    fn = pl.kernel(
        _sc_attend_body,
        mesh=mesh,
        compiler_params=pltpu.CompilerParams(
            use_tc_tiling_on_sc=False, needs_layout_passes=False),
        out_type=jax.ShapeDtypeStruct((_ITEMS * _DH,), jnp.float32),
        scratch_types=[
            pltpu.VMEM((_BLK * _ROW,), jnp.float32),
            pltpu.VMEM((_LANES, _DH), jnp.float32),
            pltpu.VMEM((_LANES,), jnp.int32),
            pltpu.VMEM((_BLK * _DH,), jnp.float32),
            pltpu.SemaphoreType.DMA,
        ],
    )
    return fn(sall, vtab)


def _rep_perms(skv):
    perm1 = np.arange(_CHUNK // 2, skv, _CHUNK)
    perm2 = np.concatenate([np.arange(o, skv, _CHUNK) for o in (8, 24)])
    perm3 = np.concatenate([np.arange(o, skv, _CHUNK) for o in (4, 12, 20, 28)])
    perm4 = np.concatenate(
        [np.arange(o, skv, _CHUNK) for o in (2, 6, 10, 14, 18, 22, 26, 30)])
    perm5 = np.concatenate([np.arange(o, skv, _CHUNK) for o in range(1, 32, 2)])
    return perm1, perm2, perm3, perm4, perm5


def kernel(x, Wq, bq, Wk, bk, Wv, bv, Wo, bo):
    B, S, D = x.shape
    H = _N_HEAD
    dh = D // H
    x2 = x.reshape(B * S, D)

    wcat = jnp.concatenate([Wq, Wk, Wv], axis=0).T
    bcat = jnp.concatenate([bq, bk, bv], axis=0)
    qkv = _mm_bias(x2, wcat, bcat, 256, 512)
    q = qkv[:, :D].reshape(S, H, dh).transpose(1, 0, 2)
    k = qkv[:, D:2 * D].reshape(S, H, dh).transpose(1, 0, 2)
    v = qkv[:, 2 * D:].reshape(S, H, dh).transpose(1, 0, 2)

    p1, p2, p3, p4, p5 = _rep_perms(S)
    ksel = jnp.concatenate(
        [k[:, p5, :], k[:, p5 - 1, :], k[:, p4, :], k[:, p3, :],
         k[:, p2, :], k[:, p1, :],
         jnp.zeros((H, _ROW - 3008, dh), jnp.float32)], axis=1)
    vtab = jnp.concatenate([v[:, p5, :], v[:, p5 - 1, :]], axis=1)

    sall = _scores(q, ksel).reshape(_ITEMS * _ROW)
    ctx = _sc_attend(sall, vtab.reshape(_ITEMS, dh))

    ctx2 = ctx.reshape(H, S, dh).transpose(1, 0, 2).reshape(S, D)
    out = _mm_bias(ctx2, Wo.T, bo, 256, 512)
    return out.reshape(B, S, D)


# speedup vs baseline: 11.0752x; 1.0024x over previous
import functools
import math

import jax
import jax.numpy as jnp
import numpy as np
from jax import lax
from jax.experimental import pallas as pl
from jax.experimental.pallas import tpu as pltpu
from jax.experimental.pallas import tpu_sc as plsc

_N_HEAD = 16
_SEQ = 2048
_DH = 64
_CHUNK = 32
_TOPK = 8
_LANES = 16

_ROW = 3072
_OFF5, _OFF5P, _OFF4, _OFF3, _OFF2, _OFF1 = 0, 1024, 2048, 2560, 2816, 2944

_NW = 32
_ITEMS = _N_HEAD * _SEQ
_PER_W = _ITEMS // _NW
_BLK = 8


def _bf(a):
    return a.astype(jnp.bfloat16)


def _mm_bias_body(x_ref, w_ref, b_ref, o_ref):
    acc = lax.dot_general(
        _bf(x_ref[...]), _bf(w_ref[...]), (((1,), (0,)), ((), ())),
        preferred_element_type=jnp.float32)
    o_ref[...] = acc + b_ref[...]


def _mm_bias(x, wt, b, bm, bn):
    M, K = x.shape
    N = wt.shape[1]
    grid = (M // bm, N // bn)
    return pl.pallas_call(
        _mm_bias_body,
        grid=grid,
        in_specs=[
            pl.BlockSpec((bm, K), lambda i, j: (i, 0)),
            pl.BlockSpec((K, bn), lambda i, j: (0, j)),
            pl.BlockSpec((1, bn), lambda i, j: (0, j)),
        ],
        out_specs=pl.BlockSpec((bm, bn), lambda i, j: (i, j)),
        out_shape=jax.ShapeDtypeStruct((M, N), jnp.float32),
    )(x, wt, b.reshape(1, N))


def _score_body(q_ref, k_ref, o_ref):
    o_ref[0] = lax.dot_general(
        _bf(q_ref[0]), _bf(k_ref[0]), (((1,), (1,)), ((), ())),
        preferred_element_type=jnp.float32)


def _scores(q, ksel, qt=256):
    H, S, dh = q.shape
    n = ksel.shape[1]
    return pl.pallas_call(
        _score_body,
        grid=(H, S // qt),
        in_specs=[
            pl.BlockSpec((1, qt, dh), lambda h, t: (h, t, 0)),
            pl.BlockSpec((1, n, dh), lambda h, t: (h, 0, 0)),
        ],
        out_specs=pl.BlockSpec((1, qt, n), lambda h, t: (h, t, 0)),
        out_shape=jax.ShapeDtypeStruct((H, S, n), jnp.float32),
    )(q, ksel)


def _lane():
    return lax.iota(jnp.int32, _LANES)


def _splat(v):
    return jnp.full((_LANES,), v, jnp.int32)


def _perm(x, idx):
    return lax.gather(
        x, idx.reshape(_LANES, 1),
        lax.GatherDimensionNumbers(
            offset_dims=(), collapsed_slice_dims=(0,), start_index_map=(0,)),
        (1,), mode=lax.GatherScatterMode.PROMISE_IN_BOUNDS)


def _merge8(sa, ia, sb, ib):
    lane = _lane()
    low = lane < 8
    l8 = lane & 7
    s = jnp.where(low, sa, _perm(sb, l8))
    i = jnp.where(low, ia, _perm(ib, l8))
    return plsc.sort_key_val(s, i, descending=True)


def _rbf(x):
    u = lax.bitcast_convert_type(x, jnp.int32)
    r = (u + 32767 + ((u >> 16) & 1)) & jnp.int32(-65536)
    return lax.bitcast_convert_type(r, jnp.float32)


def _children(p):
    lane = _lane()
    base = (p & 63) | ((p >> 6) << 7)
    return _perm(base, lane & 7) + jnp.where(lane < 8, 0, 64)


def _sc_attend_body(sall, vtab, out, sbuf, rows, idxv, obuf, sem):
    wid = lax.axis_index("s") * 2 + lax.axis_index("c")
    head = wid // 2
    base = wid * _PER_W
    lane = _lane()
    low = lane < 8
    l8 = lane & 7

    def blk_body(b, carry):
        it0 = base + b * _BLK
        pltpu.sync_copy(sall.at[pl.ds(it0 * _ROW, _BLK * _ROW)], sbuf)

        def item_body(i, carry2):
            rbase = i * _ROW

            svs = []
            for kk in range(4):
                s = plsc.load_gather(sbuf, [_splat(rbase + _OFF1 + 16 * kk) + lane])
                svs.append(plsc.sort_key_val(s, _splat(16 * kk) + lane,
                                             descending=True))
            s01 = _merge8(*svs[0], *svs[1])
            s23 = _merge8(*svs[2], *svs[3])
            _, p = _merge8(*s01, *s23)

            for off in (_OFF2, _OFF3, _OFF4, _OFF5):
                cand = _children(p)
                s = plsc.load_gather(sbuf, [_splat(rbase + off) + cand])
                sv, p = plsc.sort_key_val(s, cand, descending=True)

            part = plsc.load_gather(sbuf, [_splat(rbase + _OFF5P) + p])
            fs = jnp.where(low, sv, _perm(part, l8))
            fidx = jnp.where(low, p, _perm(p, l8) + 1024)

            x = fs * (1.0 / math.sqrt(_DH))
            e = jnp.exp(x - jnp.max(x))
            w = _rbf(e / jnp.sum(e))

            idxv[...] = fidx + head * _SEQ
            pltpu.async_copy(vtab.at[idxv], rows, sem).wait()

            acc = [jnp.zeros((_LANES,), jnp.float32) for _ in range(4)]
            for r in range(16):
                wr = _perm(w, _splat(r))
                for g in range(4):
                    v = rows[r, pl.ds(16 * g, 16)]
                    acc[g] = acc[g] + v * wr
            for g in range(4):
                plsc.store_scatter(obuf, [_splat(i * _DH + 16 * g) + lane],
                                   acc[g])
            return carry2

        lax.fori_loop(0, _BLK, item_body, 0)
        pltpu.sync_copy(obuf, out.at[pl.ds(it0 * _DH, _BLK * _DH)])
        return carry

    lax.fori_loop(0, _PER_W // _BLK, blk_body, 0)


def _sc_attend(sall, vtab):
    mesh = plsc.VectorSubcoreMesh(core_axis_name="c", subcore_axis_name="s")
    fn = pl.kernel(
        _sc_attend_body,
        mesh=mesh,
        compiler_params=pltpu.CompilerParams(
            use_tc_tiling_on_sc=False, needs_layout_passes=False),
        out_type=jax.ShapeDtypeStruct((_ITEMS * _DH,), jnp.float32),
        scratch_types=[
            pltpu.VMEM((_BLK * _ROW,), jnp.float32),
            pltpu.VMEM((_LANES, _DH), jnp.float32),
            pltpu.VMEM((_LANES,), jnp.int32),
            pltpu.VMEM((_BLK * _DH,), jnp.float32),
            pltpu.SemaphoreType.DMA,
        ],
    )
    return fn(sall, vtab)


def _rep_perms(skv):
    perm1 = np.arange(_CHUNK // 2, skv, _CHUNK)
    perm2 = np.concatenate([np.arange(o, skv, _CHUNK) for o in (8, 24)])
    perm3 = np.concatenate([np.arange(o, skv, _CHUNK) for o in (4, 12, 20, 28)])
    perm4 = np.concatenate(
        [np.arange(o, skv, _CHUNK) for o in (2, 6, 10, 14, 18, 22, 26, 30)])
    perm5 = np.concatenate([np.arange(o, skv, _CHUNK) for o in range(1, 32, 2)])
    return perm1, perm2, perm3, perm4, perm5


def kernel(x, Wq, bq, Wk, bk, Wv, bv, Wo, bo):
    B, S, D = x.shape
    H = _N_HEAD
    dh = D // H
    x2 = x.reshape(B * S, D)

    wcat = jnp.concatenate([Wq, Wk, Wv], axis=0).T
    bcat = jnp.concatenate([bq, bk, bv], axis=0)
    qkv = _mm_bias(x2, wcat, bcat, 256, 512)
    q = qkv[:, :D].reshape(S, H, dh).transpose(1, 0, 2)
    k = qkv[:, D:2 * D].reshape(S, H, dh).transpose(1, 0, 2)
    v = qkv[:, 2 * D:].reshape(S, H, dh).transpose(1, 0, 2)

    p1, p2, p3, p4, p5 = _rep_perms(S)
    ksel = jnp.concatenate(
        [k[:, p5, :], k[:, p5 - 1, :], k[:, p4, :], k[:, p3, :],
         k[:, p2, :], k[:, p1, :],
         jnp.zeros((H, _ROW - 3008, dh), jnp.float32)], axis=1)
    vtab = jnp.concatenate([v[:, p5, :], v[:, p5 - 1, :]], axis=1)

    sall = _scores(q, ksel).reshape(_ITEMS * _ROW)
    vtab_r = vtab.reshape(_ITEMS, dh).astype(jnp.bfloat16).astype(jnp.float32)
    ctx = _sc_attend(sall, vtab_r)

    ctx2 = ctx.reshape(H, S, dh).transpose(1, 0, 2).reshape(S, D)
    out = _mm_bias(ctx2, Wo.T, bo, 256, 512)
    return out.reshape(B, S, D)


# speedup vs baseline: 15.2791x; 1.3796x over previous
import functools
import math

import jax
import jax.numpy as jnp
import numpy as np
from jax import lax
from jax.experimental import pallas as pl
from jax.experimental.pallas import tpu as pltpu
from jax.experimental.pallas import tpu_sc as plsc

_N_HEAD = 16
_SEQ = 2048
_DH = 64
_CHUNK = 32
_TOPK = 8
_LANES = 16

_ROW = 3072
_OFF5, _OFF5P, _OFF4, _OFF3, _OFF2, _OFF1 = 0, 1024, 2048, 2560, 2816, 2944

_NW = 32
_ITEMS = _N_HEAD * _SEQ
_PER_W = _ITEMS // _NW
_BLK = 8


def _bf(a):
    return a.astype(jnp.bfloat16)


def _mm_bias_body(x_ref, w_ref, b_ref, o_ref):
    acc = lax.dot_general(
        _bf(x_ref[...]), _bf(w_ref[...]), (((1,), (0,)), ((), ())),
        preferred_element_type=jnp.float32)
    o_ref[...] = acc + b_ref[...]


def _mm_bias(x, wt, b, bm, bn):
    M, K = x.shape
    N = wt.shape[1]
    grid = (M // bm, N // bn)
    return pl.pallas_call(
        _mm_bias_body,
        grid=grid,
        in_specs=[
            pl.BlockSpec((bm, K), lambda i, j: (i, 0)),
            pl.BlockSpec((K, bn), lambda i, j: (0, j)),
            pl.BlockSpec((1, bn), lambda i, j: (0, j)),
        ],
        out_specs=pl.BlockSpec((bm, bn), lambda i, j: (i, j)),
        out_shape=jax.ShapeDtypeStruct((M, N), jnp.float32),
    )(x, wt, b.reshape(1, N))


def _score_body(q_ref, k_ref, o_ref):
    o_ref[0] = lax.dot_general(
        _bf(q_ref[0]), _bf(k_ref[0]), (((1,), (1,)), ((), ())),
        preferred_element_type=jnp.float32)


def _scores(q, ksel, qt=256):
    H, S, dh = q.shape
    n = ksel.shape[1]
    return pl.pallas_call(
        _score_body,
        grid=(H, S // qt),
        in_specs=[
            pl.BlockSpec((1, qt, dh), lambda h, t: (h, t, 0)),
            pl.BlockSpec((1, n, dh), lambda h, t: (h, 0, 0)),
        ],
        out_specs=pl.BlockSpec((1, qt, n), lambda h, t: (h, t, 0)),
        out_shape=jax.ShapeDtypeStruct((H, S, n), jnp.float32),
    )(q, ksel)


def _lane():
    return lax.iota(jnp.int32, _LANES)


def _splat(v):
    return jnp.full((_LANES,), v, jnp.int32)


def _perm(x, idx):
    return lax.gather(
        x, idx.reshape(_LANES, 1),
        lax.GatherDimensionNumbers(
            offset_dims=(), collapsed_slice_dims=(0,), start_index_map=(0,)),
        (1,), mode=lax.GatherScatterMode.PROMISE_IN_BOUNDS)


def _merge8(sa, ia, sb, ib):
    lane = _lane()
    low = lane < 8
    l8 = lane & 7
    s = jnp.where(low, sa, _perm(sb, l8))
    i = jnp.where(low, ia, _perm(ib, l8))
    return plsc.sort_key_val(s, i, descending=True)


def _rbf(x):
    u = lax.bitcast_convert_type(x, jnp.int32)
    r = (u + 32767 + ((u >> 16) & 1)) & jnp.int32(-65536)
    return lax.bitcast_convert_type(r, jnp.float32)


def _children(p):
    lane = _lane()
    base = (p & 63) | ((p >> 6) << 7)
    return _perm(base, lane & 7) + jnp.where(lane < 8, 0, 64)


def _sc_attend_body(sall, vtab, out, sbufA, sbufB, rows0, rows1,
                    idx0, idx1, obuf, semA, semB, sem0, sem1):
    wid = lax.axis_index("s") * 2 + lax.axis_index("c")
    head = wid // 2
    base = wid * _PER_W
    nblk = _PER_W // _BLK
    lane = _lane()
    low = lane < 8
    l8 = lane & 7

    def stage(blk, buf, sem):
        return pltpu.make_async_copy(
            sall.at[pl.ds((base + blk * _BLK) * _ROW, _BLK * _ROW)], buf, sem)

    def tournament(sbuf, i):
        rbase = i * _ROW

        svs = []
        for kk in range(4):
            s = plsc.load_gather(sbuf, [_splat(rbase + _OFF1 + 16 * kk) + lane])
            svs.append(plsc.sort_key_val(s, _splat(16 * kk) + lane,
                                         descending=True))
        s01 = _merge8(*svs[0], *svs[1])
        s23 = _merge8(*svs[2], *svs[3])
        _, p = _merge8(*s01, *s23)

        for off in (_OFF2, _OFF3, _OFF4, _OFF5):
            cand = _children(p)
            s = plsc.load_gather(sbuf, [_splat(rbase + off) + cand])
            sv, p = plsc.sort_key_val(s, cand, descending=True)

        part = plsc.load_gather(sbuf, [_splat(rbase + _OFF5P) + p])
        fs = jnp.where(low, sv, _perm(part, l8))
        fidx = jnp.where(low, p, _perm(p, l8) + 1024)

        x = fs * (1.0 / math.sqrt(_DH))
        e = jnp.exp(x - jnp.max(x))
        w = _rbf(e / jnp.sum(e))
        return w, fidx

    def vgather(fidx, idxv, rows, sem):
        idxv[...] = fidx + head * _SEQ
        return pltpu.make_async_copy(vtab.at[idxv], rows, sem)

    def accum(i, w, rows):
        acc = [jnp.zeros((_LANES,), jnp.float32) for _ in range(4)]
        for r in range(16):
            wr = _perm(w, _splat(r))
            for g in range(4):
                v = rows[r, pl.ds(16 * g, 16)]
                acc[g] = acc[g] + v * wr
        for g in range(4):
            plsc.store_scatter(obuf, [_splat(i * _DH + 16 * g) + lane], acc[g])

    def process(blk, sbuf):
        def pair_body(pr, carry):
            i0 = 2 * pr
            w0, f0 = tournament(sbuf, i0)
            cp0 = vgather(f0, idx0, rows0, sem0)
            cp0.start()
            w1, f1 = tournament(sbuf, i0 + 1)
            cp1 = vgather(f1, idx1, rows1, sem1)
            cp1.start()
            cp0.wait()
            accum(i0, w0, rows0)
            cp1.wait()
            accum(i0 + 1, w1, rows1)
            return carry
        lax.fori_loop(0, _BLK // 2, pair_body, 0)
        pltpu.sync_copy(
            obuf, out.at[pl.ds((base + blk * _BLK) * _DH, _BLK * _DH)])

    stage(0, sbufA, semA).start()

    def b2_body(b2, carry):
        blk0 = b2 * 2
        stage(blk0, sbufA, semA).wait()
        stage(blk0 + 1, sbufB, semB).start()
        process(blk0, sbufA)
        stage(blk0 + 1, sbufB, semB).wait()

        @pl.when(b2 + 1 < nblk // 2)
        def _():
            stage(blk0 + 2, sbufA, semA).start()

        process(blk0 + 1, sbufB)
        return carry

    lax.fori_loop(0, nblk // 2, b2_body, 0)


def _sc_attend(sall, vtab):
    mesh = plsc.VectorSubcoreMesh(core_axis_name="c", subcore_axis_name="s")
    fn = pl.kernel(
        _sc_attend_body,
        mesh=mesh,
        compiler_params=pltpu.CompilerParams(
            use_tc_tiling_on_sc=False, needs_layout_passes=False),
        out_type=jax.ShapeDtypeStruct((_ITEMS * _DH,), jnp.float32),
        scratch_types=[
            pltpu.VMEM((_BLK * _ROW,), jnp.float32),
            pltpu.VMEM((_BLK * _ROW,), jnp.float32),
            pltpu.VMEM((_LANES, _DH), jnp.float32),
            pltpu.VMEM((_LANES, _DH), jnp.float32),
            pltpu.VMEM((_LANES,), jnp.int32),
            pltpu.VMEM((_LANES,), jnp.int32),
            pltpu.VMEM((_BLK * _DH,), jnp.float32),
            pltpu.SemaphoreType.DMA,
            pltpu.SemaphoreType.DMA,
            pltpu.SemaphoreType.DMA,
            pltpu.SemaphoreType.DMA,
        ],
    )
    return fn(sall, vtab)


def _rep_perms(skv):
    perm1 = np.arange(_CHUNK // 2, skv, _CHUNK)
    perm2 = np.concatenate([np.arange(o, skv, _CHUNK) for o in (8, 24)])
    perm3 = np.concatenate([np.arange(o, skv, _CHUNK) for o in (4, 12, 20, 28)])
    perm4 = np.concatenate(
        [np.arange(o, skv, _CHUNK) for o in (2, 6, 10, 14, 18, 22, 26, 30)])
    perm5 = np.concatenate([np.arange(o, skv, _CHUNK) for o in range(1, 32, 2)])
    return perm1, perm2, perm3, perm4, perm5


def kernel(x, Wq, bq, Wk, bk, Wv, bv, Wo, bo):
    B, S, D = x.shape
    H = _N_HEAD
    dh = D // H
    x2 = x.reshape(B * S, D)

    wcat = jnp.concatenate([Wq, Wk, Wv], axis=0).T
    bcat = jnp.concatenate([bq, bk, bv], axis=0)
    qkv = _mm_bias(x2, wcat, bcat, 256, 512)
    q = qkv[:, :D].reshape(S, H, dh).transpose(1, 0, 2)
    k = qkv[:, D:2 * D].reshape(S, H, dh).transpose(1, 0, 2)
    v = qkv[:, 2 * D:].reshape(S, H, dh).transpose(1, 0, 2)

    p1, p2, p3, p4, p5 = _rep_perms(S)
    ksel = jnp.concatenate(
        [k[:, p5, :], k[:, p5 - 1, :], k[:, p4, :], k[:, p3, :],
         k[:, p2, :], k[:, p1, :],
         jnp.zeros((H, _ROW - 3008, dh), jnp.float32)], axis=1)
    vtab = jnp.concatenate([v[:, p5, :], v[:, p5 - 1, :]], axis=1)

    sall = _scores(q, ksel).reshape(_ITEMS * _ROW)
    vtab_r = vtab.reshape(_ITEMS, dh).astype(jnp.bfloat16).astype(jnp.float32)
    ctx = _sc_attend(sall, vtab_r)

    ctx2 = ctx.reshape(H, S, dh).transpose(1, 0, 2).reshape(S, D)
    out = _mm_bias(ctx2, Wo.T, bo, 256, 512)
    return out.reshape(B, S, D)
